# baseline (speedup 1.0000x reference)
import functools

import jax
import jax.numpy as jnp
from jax import lax
from jax.experimental import pallas as pl
from jax.experimental.pallas import tpu as pltpu
from jax.experimental.pallas import tpu_sc as plsc

_FIELD_NUM = 26
_EMBED_DIM = 64
_EMBED_SIZE = _FIELD_NUM * _EMBED_DIM
_IDX_PAD = 512
_ROW_BLK = 104
_N_BLK = _EMBED_SIZE // _ROW_BLK
_K_EARLY = 10
_NC, _NS = 2, 16
_NW = _NC * _NS
_ROWS_PER_W = _EMBED_SIZE // _NW
_LANES = 128
_ONES_PAD = 64


def _sc_mask_body(idx_hbm, ones_hbm, mask_hbm, idx_v, ones_v, buf_v):
    w = lax.axis_index("s") * _NC + lax.axis_index("c")
    row_lo = w * _ROWS_PER_W
    pltpu.sync_copy(idx_hbm, idx_v)
    pltpu.sync_copy(ones_hbm, ones_v)
    ones = jnp.full((16,), 1.0, dtype=jnp.float32)
    zeros = jnp.zeros((16,), dtype=jnp.float32)
    for j in range(_ONES_PAD // 16):
        grp = ones_v[pl.ds(j * 16, 16)]
        plsc.store_scatter(buf_v, [grp], ones)
    lo = jnp.full((16,), row_lo, dtype=jnp.int32)
    for j in range(_IDX_PAD // 16):
        grp = idx_v[pl.ds(j * 16, 16)]
        local = grp - lo
        m = (local >= 0) & (local < _ROWS_PER_W)
        safe = jnp.where(m, local, 0) * _LANES
        plsc.store_scatter(buf_v, [safe], zeros, mask=m)
    pltpu.sync_copy(
        buf_v, mask_hbm.at[pl.ds(row_lo * _LANES, _ROWS_PER_W * _LANES)])


def _tc_early_body(idx_ref, x_ref, o_ref):
    i = pl.program_id(0)
    ids = idx_ref[...]
    pos = jax.lax.broadcasted_iota(jnp.int32, (_ROW_BLK, _IDX_PAD), 0) + i * _ROW_BLK
    hit = jnp.any(pos == ids, axis=1, keepdims=True)
    o_ref[...] = jnp.where(hit, 0.0, x_ref[...])


def _tc_late_body(prev_ref, mask_ref, x_ref, o_ref):
    del prev_ref
    o_ref[...] = x_ref[...] * mask_ref[:, 0:1]


def kernel(embed, embed_ele_indices):
    B = embed.shape[0]
    x_t = embed.transpose(1, 2, 0).reshape(_EMBED_SIZE, B)
    idx = embed_ele_indices.astype(jnp.int32)
    idx_pad = jnp.concatenate(
        [idx, jnp.broadcast_to(idx[0:1], (_IDX_PAD - idx.shape[0],))])
    sentinel = jnp.full((_IDX_PAD - idx.shape[0],), 2**30, dtype=jnp.int32)
    idx_sent = jnp.concatenate([idx, sentinel]).reshape(1, _IDX_PAD)
    ones_pos = jnp.concatenate([
        jnp.arange(_ROWS_PER_W, dtype=jnp.int32) * _LANES,
        jnp.zeros((_ONES_PAD - _ROWS_PER_W,), dtype=jnp.int32),
    ])

    mesh = plsc.VectorSubcoreMesh(core_axis_name="c", subcore_axis_name="s")
    sc_mask = functools.partial(
        pl.kernel,
        mesh=mesh,
        out_type=jax.ShapeDtypeStruct((_EMBED_SIZE * _LANES,), jnp.float32),
        scratch_types=[
            pltpu.VMEM((_IDX_PAD,), jnp.int32),
            pltpu.VMEM((_ONES_PAD,), jnp.int32),
            pltpu.VMEM((_ROWS_PER_W * _LANES,), jnp.float32),
        ],
        compiler_params=pltpu.CompilerParams(needs_layout_passes=False),
    )(_sc_mask_body)
    mask2d = sc_mask(idx_pad, ones_pos).reshape(_EMBED_SIZE, _LANES)

    early = pl.pallas_call(
        _tc_early_body,
        grid=(_K_EARLY,),
        in_specs=[
            pl.BlockSpec((1, _IDX_PAD), lambda i: (0, 0)),
            pl.BlockSpec((_ROW_BLK, B), lambda i: (i, 0)),
        ],
        out_specs=pl.BlockSpec((_ROW_BLK, B), lambda i: (i, 0)),
        out_shape=jax.ShapeDtypeStruct((_EMBED_SIZE, B), jnp.float32),
    )(idx_sent, x_t)

    out_t = pl.pallas_call(
        _tc_late_body,
        grid=(_N_BLK - _K_EARLY,),
        in_specs=[
            pl.BlockSpec(memory_space=pltpu.MemorySpace.HBM),
            pl.BlockSpec((_ROW_BLK, _LANES), lambda i: (i + _K_EARLY, 0)),
            pl.BlockSpec((_ROW_BLK, B), lambda i: (i + _K_EARLY, 0)),
        ],
        out_specs=pl.BlockSpec((_ROW_BLK, B), lambda i: (i + _K_EARLY, 0)),
        out_shape=jax.ShapeDtypeStruct((_EMBED_SIZE, B), jnp.float32),
        input_output_aliases={0: 0},
    )(early, mask2d, x_t)
    return out_t.reshape(_FIELD_NUM, _EMBED_DIM, B).transpose(2, 0, 1)


# speedup vs baseline: 1.3126x; 1.3126x over previous
import functools

import jax
import jax.numpy as jnp
from jax import lax
from jax.experimental import pallas as pl
from jax.experimental.pallas import tpu as pltpu
from jax.experimental.pallas import tpu_sc as plsc

_FIELD_NUM = 26
_EMBED_DIM = 64
_EMBED_SIZE = _FIELD_NUM * _EMBED_DIM
_IDX_PAD = 512
_ROW_BLK = 208
_N_BLK = _EMBED_SIZE // _ROW_BLK
_K_EARLY = 6
_NC, _NS = 2, 16
_NW = _NC * _NS
_ROWS_PER_W = _EMBED_SIZE // _NW
_LANES = 128
_ONES_PAD = 64


def _sc_mask_body(idx_hbm, ones_hbm, mask_hbm, idx_v, ones_v, buf_v):
    w = lax.axis_index("s") * _NC + lax.axis_index("c")
    row_lo = w * _ROWS_PER_W
    pltpu.sync_copy(idx_hbm, idx_v)
    pltpu.sync_copy(ones_hbm, ones_v)
    ones = jnp.full((16,), 1.0, dtype=jnp.float32)
    zeros = jnp.zeros((16,), dtype=jnp.float32)
    for j in range(_ONES_PAD // 16):
        grp = ones_v[pl.ds(j * 16, 16)]
        plsc.store_scatter(buf_v, [grp], ones)
    lo = jnp.full((16,), row_lo, dtype=jnp.int32)
    for j in range(_IDX_PAD // 16):
        grp = idx_v[pl.ds(j * 16, 16)]
        local = grp - lo
        m = (local >= 0) & (local < _ROWS_PER_W)
        safe = jnp.where(m, local, 0) * _LANES
        plsc.store_scatter(buf_v, [safe], zeros, mask=m)
    pltpu.sync_copy(
        buf_v, mask_hbm.at[pl.ds(row_lo * _LANES, _ROWS_PER_W * _LANES)])


def _tc_early_body(idx_ref, x_ref, o_ref):
    i = pl.program_id(0)
    ids = idx_ref[...]
    pos = jax.lax.broadcasted_iota(jnp.int32, (_ROW_BLK, _IDX_PAD), 0) + i * _ROW_BLK
    hit = jnp.any(pos == ids, axis=1, keepdims=True)
    o_ref[...] = jnp.where(hit, 0.0, x_ref[...])


def _tc_late_body(prev_ref, idx_ref, x_ref, o_ref):
    del prev_ref
    i = pl.program_id(0) + _K_EARLY
    ids = idx_ref[...]
    pos = jax.lax.broadcasted_iota(jnp.int32, (_ROW_BLK, _IDX_PAD), 0) + i * _ROW_BLK
    hit = jnp.any(pos == ids, axis=1, keepdims=True)
    o_ref[...] = jnp.where(hit, 0.0, x_ref[...])


def kernel(embed, embed_ele_indices):
    B = embed.shape[0]
    x_t = embed.transpose(1, 2, 0).reshape(_EMBED_SIZE, B)
    idx = embed_ele_indices.astype(jnp.int32)
    idx_pad = jnp.concatenate(
        [idx, jnp.broadcast_to(idx[0:1], (_IDX_PAD - idx.shape[0],))])
    sentinel = jnp.full((_IDX_PAD - idx.shape[0],), 2**30, dtype=jnp.int32)
    idx_sent = jnp.concatenate([idx, sentinel]).reshape(1, _IDX_PAD)
    ones_pos = jnp.concatenate([
        jnp.arange(_ROWS_PER_W, dtype=jnp.int32) * _LANES,
        jnp.zeros((_ONES_PAD - _ROWS_PER_W,), dtype=jnp.int32),
    ])

    mesh = plsc.VectorSubcoreMesh(core_axis_name="c", subcore_axis_name="s")
    sc_mask = functools.partial(
        pl.kernel,
        mesh=mesh,
        out_type=jax.ShapeDtypeStruct((_EMBED_SIZE * _LANES,), jnp.float32),
        scratch_types=[
            pltpu.VMEM((_IDX_PAD,), jnp.int32),
            pltpu.VMEM((_ONES_PAD,), jnp.int32),
            pltpu.VMEM((_ROWS_PER_W * _LANES,), jnp.float32),
        ],
        compiler_params=pltpu.CompilerParams(needs_layout_passes=False),
    )(_sc_mask_body)
    mask2d = sc_mask(idx_pad, ones_pos).reshape(_EMBED_SIZE, _LANES)

    early = pl.pallas_call(
        _tc_early_body,
        grid=(_K_EARLY,),
        in_specs=[
            pl.BlockSpec((1, _IDX_PAD), lambda i: (0, 0)),
            pl.BlockSpec((_ROW_BLK, B), lambda i: (i, 0)),
        ],
        out_specs=pl.BlockSpec((_ROW_BLK, B), lambda i: (i, 0)),
        out_shape=jax.ShapeDtypeStruct((_EMBED_SIZE, B), jnp.float32),
    )(idx_sent, x_t)

    out_t = pl.pallas_call(
        _tc_late_body,
        grid=(_N_BLK - _K_EARLY,),
        in_specs=[
            pl.BlockSpec(memory_space=pltpu.MemorySpace.HBM),
            pl.BlockSpec((1, _IDX_PAD), lambda i: (0, 0)),
            pl.BlockSpec((_ROW_BLK, B), lambda i: (i + _K_EARLY, 0)),
        ],
        out_specs=pl.BlockSpec((_ROW_BLK, B), lambda i: (i + _K_EARLY, 0)),
        out_shape=jax.ShapeDtypeStruct((_EMBED_SIZE, B), jnp.float32),
        input_output_aliases={0: 0},
    )(early, idx_sent, x_t)
    return out_t.reshape(_FIELD_NUM, _EMBED_DIM, B).transpose(2, 0, 1)
